# baseline (speedup 1.0000x reference)
import jax
import jax.numpy as jnp
from jax.experimental import pallas as pl
from jax.experimental.pallas import tpu as pltpu

_N = 4096
_D = 64
_BM = 512
_NCHUNK = 8
_NBUF = 6


def _mm_kernel(adj_hbm, emb_ref, out_ref, *scratch):
    bufs = scratch[:_NBUF]
    sems = scratch[_NBUF:]

    def copy(ci):
        return pltpu.make_async_copy(
            adj_hbm.at[pl.ds(ci * _BM, _BM), :],
            bufs[ci % _NBUF], sems[ci],
        )

    for ci in range(_NBUF):
        copy(ci).start()
    for ci in range(_NCHUNK):
        copy(ci).wait()
        out_ref[pl.ds(ci * _BM, _BM), :] = jnp.dot(
            bufs[ci % _NBUF][...], emb_ref[...],
            preferred_element_type=jnp.float32,
        )
        if ci < _NCHUNK - _NBUF:
            copy(ci + _NBUF).start()


def kernel(adj, embeds):
    return pl.pallas_call(
        _mm_kernel,
        in_specs=[
            pl.BlockSpec(memory_space=pl.ANY),
            pl.BlockSpec(memory_space=pltpu.MemorySpace.VMEM),
        ],
        out_specs=pl.BlockSpec(memory_space=pltpu.MemorySpace.VMEM),
        out_shape=jax.ShapeDtypeStruct((_N, _D), jnp.float32),
        scratch_shapes=(
            [pltpu.VMEM((_BM, _N), jnp.float32) for _ in range(_NBUF)]
            + [pltpu.SemaphoreType.DMA for _ in range(_NCHUNK)]
        ),
    )(adj, embeds)


# speedup vs baseline: 1.0821x; 1.0821x over previous
import jax
import jax.numpy as jnp
from jax.experimental import pallas as pl
from jax.experimental.pallas import tpu as pltpu

_N = 4096
_D = 64
_BM = 512


def _matmul_kernel(adj_ref, emb_ref, out_ref):
    out_ref[...] = jnp.dot(
        adj_ref[...], emb_ref[...], preferred_element_type=jnp.float32
    )


def kernel(adj, embeds):
    return pl.pallas_call(
        _matmul_kernel,
        grid=(_N // _BM,),
        in_specs=[
            pl.BlockSpec((_BM, _N), lambda i: (i, 0)),
            pl.BlockSpec(memory_space=pltpu.MemorySpace.VMEM),
        ],
        out_specs=pl.BlockSpec((_BM, _D), lambda i: (i, 0)),
        out_shape=jax.ShapeDtypeStruct((_N, _D), jnp.float32),
    )(adj, embeds)
